# Initial kernel scaffold; baseline (speedup 1.0000x reference)
import jax
import jax.numpy as jnp
from jax.experimental import pallas as pl


def kernel(x, target):
    raise NotImplementedError("write your pallas kernel here")


# speedup vs baseline: 5.5869x; 5.5869x over previous
import math

import jax
import jax.numpy as jnp
from jax.experimental import pallas as pl
from jax.experimental.pallas import tpu as pltpu

_SIZE = 32000
_PADDING_IDX = 0
_SMOOTHING = 0.1
_CONFIDENCE = 1.0 - _SMOOTHING
_EPS = _SMOOTHING / (_SIZE - 2)
_C_ROW = (_SIZE - 2) * _EPS * math.log(_EPS) + _CONFIDENCE * math.log(_CONFIDENCE)

_BN = 256
_BV = 3200


def _loss_kernel(t_ref, x_ref, o_ref, acc_ref, cnt_ref):
    i = pl.program_id(0)
    j = pl.program_id(1)
    nbi = pl.num_programs(0)
    nbj = pl.num_programs(1)

    @pl.when(jnp.logical_and(i == 0, j == 0))
    def _init():
        acc_ref[0] = 0.0
        cnt_ref[0] = 0.0

    xb = x_ref[...]
    tb = t_ref[0, 0, :]
    nonpad = (tb != _PADDING_IDX).astype(jnp.float32)

    rowsum = jnp.sum(xb, axis=1)
    s = _EPS * jnp.sum(rowsum * nonpad)

    cols = jax.lax.broadcasted_iota(jnp.int32, (_BN, _BV), 1) + j * _BV
    hit = (cols == tb[:, None])
    xt = jnp.sum(jnp.where(hit, xb, 0.0), axis=1)
    s = s + (_CONFIDENCE - _EPS) * jnp.sum(xt * nonpad)

    @pl.when(j == 0)
    def _col0():
        acc_ref[0] += s - _EPS * jnp.sum(xb[:, 0] * nonpad)
        cnt_ref[0] += jnp.sum(nonpad)

    @pl.when(j != 0)
    def _rest():
        acc_ref[0] += s

    @pl.when(jnp.logical_and(i == nbi - 1, j == nbj - 1))
    def _final():
        n_rows = _BN * nbi
        o_ref[0, 0] = (cnt_ref[0] * _C_ROW - acc_ref[0]) / n_rows


def kernel(x, target):
    n, v = x.shape
    nbi = n // _BN
    nbj = v // _BV
    t3 = target.astype(jnp.int32).reshape(nbi, 1, _BN)
    out = pl.pallas_call(
        _loss_kernel,
        grid=(nbi, nbj),
        in_specs=[
            pl.BlockSpec((1, 1, _BN), lambda i, j: (i, 0, 0)),
            pl.BlockSpec((_BN, _BV), lambda i, j: (i, j)),
        ],
        out_specs=pl.BlockSpec(
            (1, 1), lambda i, j: (0, 0), memory_space=pltpu.SMEM
        ),
        out_shape=jax.ShapeDtypeStruct((1, 1), jnp.float32),
        scratch_shapes=[
            pltpu.SMEM((1,), jnp.float32),
            pltpu.SMEM((1,), jnp.float32),
        ],
        compiler_params=pltpu.CompilerParams(
            dimension_semantics=("arbitrary", "arbitrary"),
        ),
    )(t3, x)
    return out[0, 0]


# speedup vs baseline: 8.4920x; 1.5200x over previous
import math

import jax
import jax.numpy as jnp
from jax.experimental import pallas as pl
from jax.experimental.pallas import tpu as pltpu

_SIZE = 32000
_PADDING_IDX = 0
_SMOOTHING = 0.1
_CONFIDENCE = 1.0 - _SMOOTHING
_EPS = _SMOOTHING / (_SIZE - 2)
_C_ROW = (_SIZE - 2) * _EPS * math.log(_EPS) + _CONFIDENCE * math.log(_CONFIDENCE)

_BN = 512
_BV = 6400


def _loss_kernel(t_ref, x_ref, o_ref, acc_ref, cnt_ref):
    i = pl.program_id(0)
    j = pl.program_id(1)
    nbi = pl.num_programs(0)
    nbj = pl.num_programs(1)

    @pl.when(jnp.logical_and(i == 0, j == 0))
    def _init():
        acc_ref[0] = 0.0
        cnt_ref[0] = 0.0

    xb = x_ref[...]
    tb = t_ref[0, 0, :]
    nonpad = (tb != _PADDING_IDX).astype(jnp.float32)

    rowsum = jnp.sum(xb, axis=1)
    s = _EPS * jnp.sum(rowsum * nonpad)

    cols = jax.lax.broadcasted_iota(jnp.int32, (_BN, _BV), 1) + j * _BV
    hit = (cols == tb[:, None])
    xt = jnp.sum(jnp.where(hit, xb, 0.0), axis=1)
    s = s + (_CONFIDENCE - _EPS) * jnp.sum(xt * nonpad)

    @pl.when(j == 0)
    def _col0():
        acc_ref[0] += s - _EPS * jnp.sum(xb[:, 0] * nonpad)
        cnt_ref[0] += jnp.sum(nonpad)

    @pl.when(j != 0)
    def _rest():
        acc_ref[0] += s

    @pl.when(jnp.logical_and(i == nbi - 1, j == nbj - 1))
    def _final():
        n_rows = _BN * nbi
        o_ref[0, 0] = (cnt_ref[0] * _C_ROW - acc_ref[0]) / n_rows


def kernel(x, target):
    n, v = x.shape
    nbi = n // _BN
    nbj = v // _BV
    t3 = target.astype(jnp.int32).reshape(nbi, 1, _BN)
    out = pl.pallas_call(
        _loss_kernel,
        grid=(nbi, nbj),
        in_specs=[
            pl.BlockSpec((1, 1, _BN), lambda i, j: (i, 0, 0)),
            pl.BlockSpec((_BN, _BV), lambda i, j: (i, j)),
        ],
        out_specs=pl.BlockSpec(
            (1, 1), lambda i, j: (0, 0), memory_space=pltpu.SMEM
        ),
        out_shape=jax.ShapeDtypeStruct((1, 1), jnp.float32),
        scratch_shapes=[
            pltpu.SMEM((1,), jnp.float32),
            pltpu.SMEM((1,), jnp.float32),
        ],
        compiler_params=pltpu.CompilerParams(
            dimension_semantics=("arbitrary", "arbitrary"),
        ),
    )(t3, x)
    return out[0, 0]
